# baseline (speedup 1.0000x reference)
import functools

import jax
import jax.numpy as jnp
from jax import lax
from jax.experimental import pallas as pl
from jax.experimental.pallas import tpu as pltpu
from jax.experimental.pallas import tpu_sc as plsc

NUM_USER = 10000
NUM_ITEM = 40000
N = NUM_USER + NUM_ITEM
E = 500000
EU = 50000
D_FEAT = 128
D_LAT = 64
B = 1024

NC = 2
NS = 16
CH = 128

C1 = 64
GRP = 6
SEG1_CHUNKS = 978
SEG1_PT = SEG1_CHUNKS * C1
SEG1_HALF = 8 * SEG1_PT
SEG1_PAIRS = 81
TROW = 50000
TAB_ROWS = 52000
ACC_ROWS = 50048
ACC_PT = ACC_ROWS // NS

SEG2_CHUNKS = -(-EU // (NS * CH))
SEG2_PT = SEG2_CHUNKS * CH
SEG2_PAD = NS * SEG2_PT
UACC_ROWS = 10048
UACC_PT = UACC_ROWS // NS
BPT = B // NS


def _dense_body(vp, ap, tp, vf, af, tf, vw, aw, tw, vb, ab, tb,
                vcw, acw, tcw, out_lo, out_hi):
    pid = pl.program_id(0)

    def emit(h):
        out_lo[...] = h[:, :32]
        out_hi[...] = h[:, 32:]

    @pl.when(pid < 5)
    def _():
        h = None
        for p_ref, cw_ref in ((vp, vcw), (ap, acw), (tp, tcw)):
            x = p_ref[...]
            nrm = jnp.sqrt(jnp.sum(x * x, axis=1, keepdims=True))
            xn = x / jnp.maximum(nrm, 1e-12)
            hm = jnp.dot(xn, cw_ref[...], preferred_element_type=jnp.float32)
            h = hm if h is None else h + hm
        emit(h)

    @pl.when(pid >= 5)
    def _():
        h = None
        for f_ref, w_ref, b_ref, cw_ref in (
                (vf, vw, vb, vcw), (af, aw, ab, acw), (tf, tw, tb, tcw)):
            t = lax.dot_general(f_ref[...], w_ref[...],
                                (((1,), (1,)), ((), ())),
                                preferred_element_type=jnp.float32)
            t = t + b_ref[...]
            nrm = jnp.sqrt(jnp.sum(t * t, axis=1, keepdims=True))
            xn = t / jnp.maximum(nrm, 1e-12)
            hm = jnp.dot(xn, cw_ref[...], preferred_element_type=jnp.float32)
            h = hm if h is None else h + hm
        emit(h)


def _final_body(gu, pos, neg, w, pos_out, neg_out):
    ua = jnp.dot(gu[...] * (1.0 / 3.0), w[...],
                 preferred_element_type=jnp.float32)
    ut = jnp.where(ua >= 0, ua, 0.01 * ua)
    pos_out[...] = jnp.sum(ut * pos[...], axis=1) * (1.0 / 3.0)
    neg_out[...] = jnp.sum(ut * neg[...], axis=1) * (1.0 / 3.0)


def _seg1_kernel(t_lo, t_hi, ucol, icol, zrows,
                 rep_lo, rep_hi, sidx_a, didx_a, rows_a, gsem_a, ssem_a,
                 sidx_b, didx_b, rows_b, gsem_b, ssem_b, acc):
    c = lax.axis_index("c")
    s = lax.axis_index("s")

    pltpu.sync_copy(zrows, acc.at[pl.ds(s * ACC_PT, ACC_PT)])
    plsc.subcore_barrier()

    def run_edges(tab, sarr, darr):
        cbase = (s % 8) * SEG1_CHUNKS

        def stage(sidx, didx, crow):
            pltpu.sync_copy(sarr.at[pl.ds(crow, GRP)], sidx)
            pltpu.sync_copy(darr.at[pl.ds(crow, GRP)], didx)

        def fire_g(sidx, rows, gsem):
            for k in range(GRP):
                pltpu.async_copy(tab.at[sidx.at[k]],
                                 rows.at[pl.ds(k * C1, C1)], gsem)

        def drain_g(sidx, rows, gsem):
            for k in range(GRP):
                pltpu.make_async_copy(tab.at[sidx.at[k]],
                                      rows.at[pl.ds(k * C1, C1)], gsem).wait()

        def fire_s(didx, rows, ssem):
            for k in range(GRP):
                pltpu.async_copy(rows.at[pl.ds(k * C1, C1)],
                                 acc.at[didx.at[k]], ssem, add=True)

        def drain_s(didx, rows, ssem):
            for k in range(GRP):
                pltpu.make_async_copy(rows.at[pl.ds(k * C1, C1)],
                                      acc.at[didx.at[k]], ssem).wait()

        npairs = SEG1_PAIRS
        stage(sidx_a, didx_a, cbase)
        fire_g(sidx_a, rows_a, gsem_a)

        def body(p, carry):
            crow_b = cbase + (2 * p + 1) * GRP
            drain_g(sidx_a, rows_a, gsem_a)

            @pl.when(p > 0)
            def _():
                drain_s(didx_b, rows_b, ssem_b)
            fire_s(didx_a, rows_a, ssem_a)
            stage(sidx_b, didx_b, crow_b)
            fire_g(sidx_b, rows_b, gsem_b)
            drain_s(didx_a, rows_a, ssem_a)

            @pl.when(p < npairs - 1)
            def _():
                stage(sidx_a, didx_a, crow_b + GRP)
                fire_g(sidx_a, rows_a, gsem_a)
            drain_g(sidx_b, rows_b, gsem_b)
            fire_s(didx_b, rows_b, ssem_b)
            return carry
        lax.fori_loop(0, npairs, body, 0)
        drain_s(didx_b, rows_b, ssem_b)
        stage(sidx_a, didx_a, cbase + 2 * npairs * GRP)
        fire_g(sidx_a, rows_a, gsem_a)
        drain_g(sidx_a, rows_a, gsem_a)
        fire_s(didx_a, rows_a, ssem_a)
        drain_s(didx_a, rows_a, ssem_a)

    @pl.when(jnp.logical_and(c == 0, s < 8))
    def _():
        run_edges(t_lo, ucol, icol)

    @pl.when(jnp.logical_and(c == 0, s >= 8))
    def _():
        run_edges(t_lo, icol, ucol)

    @pl.when(jnp.logical_and(c == 1, s < 8))
    def _():
        run_edges(t_hi, ucol, icol)

    @pl.when(jnp.logical_and(c == 1, s >= 8))
    def _():
        run_edges(t_hi, icol, ucol)

    plsc.subcore_barrier()

    row0 = s * ACC_PT

    @pl.when(c == 0)
    def _():
        pltpu.sync_copy(acc.at[pl.ds(row0, ACC_PT)],
                        rep_lo.at[pl.ds(row0, ACC_PT)])

    @pl.when(c == 1)
    def _():
        pltpu.sync_copy(acc.at[pl.ds(row0, ACC_PT)],
                        rep_hi.at[pl.ds(row0, ACC_PT)])


def _seg2_kernel(rep_lo, rep_hi, s5ix, d5ix, posix, negix, unodes, zrows,
                 uacc_lo, uacc_hi, pos_lo, pos_hi, neg_lo, neg_hi,
                 gu_lo, gu_hi,
                 sidx, didx, rows, bidx, brows, uacc):
    c = lax.axis_index("c")
    s = lax.axis_index("s")

    pltpu.sync_copy(zrows, uacc.at[pl.ds(s * UACC_PT, UACC_PT)])
    plsc.subcore_barrier()

    ebase = s * SEG2_PT

    def run(rep_tab, pos_out, neg_out, uacc_out, gu_out):
        def body(j, carry):
            off = ebase + j * CH
            pltpu.sync_copy(s5ix.at[pl.ds(off, CH)], sidx)
            pltpu.sync_copy(d5ix.at[pl.ds(off, CH)], didx)
            pltpu.sync_copy(rep_tab.at[sidx], rows)
            pltpu.sync_copy(rows, uacc.at[didx], add=True)
            return carry
        lax.fori_loop(0, SEG2_CHUNKS, body, 0)

        b0 = s * BPT
        pltpu.sync_copy(posix.at[pl.ds(b0, BPT)], bidx)
        pltpu.sync_copy(rep_tab.at[bidx], brows)
        pltpu.sync_copy(brows, pos_out.at[pl.ds(b0, BPT)])
        pltpu.sync_copy(negix.at[pl.ds(b0, BPT)], bidx)
        pltpu.sync_copy(rep_tab.at[bidx], brows)
        pltpu.sync_copy(brows, neg_out.at[pl.ds(b0, BPT)])

        plsc.subcore_barrier()
        r0 = s * UACC_PT
        pltpu.sync_copy(uacc.at[pl.ds(r0, UACC_PT)],
                        uacc_out.at[pl.ds(r0, UACC_PT)])
        plsc.subcore_barrier()
        pltpu.sync_copy(unodes.at[pl.ds(b0, BPT)], bidx)
        pltpu.sync_copy(uacc_out.at[bidx], brows)
        pltpu.sync_copy(brows, gu_out.at[pl.ds(b0, BPT)])

    @pl.when(c == 0)
    def _():
        run(rep_lo, pos_lo, neg_lo, uacc_lo, gu_lo)

    @pl.when(c == 1)
    def _():
        run(rep_hi, pos_hi, neg_hi, uacc_hi, gu_hi)


def _sc_mesh():
    return plsc.VectorSubcoreMesh(core_axis_name="c", subcore_axis_name="s",
                                  num_cores=NC, num_subcores=NS)


@functools.partial(jax.jit, static_argnames=())
def kernel(v_feat, a_feat, t_feat, v_pref, a_pref, t_pref, v_mlp_w, v_mlp_b,
           a_mlp_w, a_mlp_b, t_mlp_w, t_mlp_b, v_conv_w, a_conv_w, t_conv_w,
           user_conv_w, edge_index, user_index_5, user_nodes, pos_item_nodes,
           neg_item_nodes):
    f32 = jnp.float32
    i32 = jnp.int32

    bm = 2000
    vb2 = v_mlp_b.reshape(1, D_LAT)
    ab2 = a_mlp_b.reshape(1, D_LAT)
    tb2 = t_mlp_b.reshape(1, D_LAT)
    t_lo, t_hi = pl.pallas_call(
        _dense_body,
        grid=(TAB_ROWS // bm,),
        in_specs=[pl.BlockSpec((bm, D_LAT), lambda i: (jnp.minimum(i, 4), 0))] * 3
                 + [pl.BlockSpec((bm, D_FEAT),
                                 lambda i: (jnp.clip(i - 5, 0, 19), 0))] * 3
                 + [pl.BlockSpec((D_LAT, D_FEAT), lambda i: (0, 0))] * 3
                 + [pl.BlockSpec((1, D_LAT), lambda i: (0, 0))] * 3
                 + [pl.BlockSpec((D_LAT, D_LAT), lambda i: (0, 0))] * 3,
        out_specs=[pl.BlockSpec((bm, 32), lambda i: (i, 0))] * 2,
        out_shape=[jax.ShapeDtypeStruct((TAB_ROWS, 32), f32)] * 2,
    )(v_pref, a_pref, t_pref, v_feat, a_feat, t_feat,
      v_mlp_w, a_mlp_w, t_mlp_w, vb2, ab2, tb2, v_conv_w, a_conv_w, t_conv_w)

    ei = edge_index.astype(i32)
    cpad = jnp.full((SEG1_HALF - E,), TROW, i32)
    ucol = jnp.concatenate([ei[:, 0], cpad]).reshape(-1, C1)
    icol = jnp.concatenate([ei[:, 1], cpad]).reshape(-1, C1)

    zrows1 = jnp.zeros((ACC_PT, 32), f32)

    rep_lo, rep_hi = pl.kernel(
        _seg1_kernel,
        out_type=[jax.ShapeDtypeStruct((ACC_ROWS, 32), f32)] * 2,
        mesh=_sc_mesh(),
        compiler_params=pltpu.CompilerParams(use_tc_tiling_on_sc=False),
        scratch_types=[
            pltpu.VMEM((GRP, C1), i32),
            pltpu.VMEM((GRP, C1), i32),
            pltpu.VMEM((GRP * C1, 32), f32),
            pltpu.SemaphoreType.DMA,
            pltpu.SemaphoreType.DMA,
        ] * 2 + [
            pltpu.VMEM_SHARED((ACC_ROWS, 32), f32),
        ],
    )(t_lo, t_hi, ucol, icol, zrows1)

    s5 = user_index_5[0].astype(i32)
    d5 = user_index_5[1].astype(i32)
    pad2 = SEG2_PAD - EU
    s5ix = jnp.concatenate([s5, jnp.zeros((pad2,), i32)])
    d5ix = jnp.concatenate([d5, jnp.full((pad2,), NUM_USER, i32)])
    zrows2 = jnp.zeros((UACC_PT, 32), f32)

    (uacc_lo, uacc_hi, pos_lo, pos_hi, neg_lo, neg_hi, gu_lo, gu_hi
     ) = pl.kernel(
        _seg2_kernel,
        out_type=[jax.ShapeDtypeStruct((UACC_ROWS, 32), f32)] * 2
                 + [jax.ShapeDtypeStruct((B, 32), f32)] * 6,
        mesh=_sc_mesh(),
        compiler_params=pltpu.CompilerParams(use_tc_tiling_on_sc=False),
        scratch_types=[
            pltpu.VMEM((CH,), i32),
            pltpu.VMEM((CH,), i32),
            pltpu.VMEM((CH, 32), f32),
            pltpu.VMEM((BPT,), i32),
            pltpu.VMEM((BPT, 32), f32),
            pltpu.VMEM_SHARED((UACC_ROWS, 32), f32),
        ],
    )(rep_lo, rep_hi, s5ix, d5ix,
      pos_item_nodes.astype(i32), neg_item_nodes.astype(i32),
      user_nodes.astype(i32), zrows2)

    del uacc_lo, uacc_hi

    gu = jnp.concatenate([gu_lo, gu_hi], axis=1)
    pos = jnp.concatenate([pos_lo, pos_hi], axis=1)
    neg = jnp.concatenate([neg_lo, neg_hi], axis=1)

    pos_scores, neg_scores = pl.pallas_call(
        _final_body,
        out_shape=[jax.ShapeDtypeStruct((B,), f32)] * 2,
    )(gu, pos, neg, user_conv_w)

    return (pos_scores, neg_scores)


# speedup vs baseline: 1.0629x; 1.0629x over previous
import functools

import jax
import jax.numpy as jnp
from jax import lax
from jax.experimental import pallas as pl
from jax.experimental.pallas import tpu as pltpu
from jax.experimental.pallas import tpu_sc as plsc

NUM_USER = 10000
NUM_ITEM = 40000
N = NUM_USER + NUM_ITEM
E = 500000
EU = 50000
D_FEAT = 128
D_LAT = 64
B = 1024

NC = 2
NS = 16
CH = 128

GRP = 3
SEG1_CHUNKS = 489
SEG1_PT = SEG1_CHUNKS * CH
SEG1_HALF = 8 * SEG1_PT
SEG1_PAIRS = 81
TROW = 50000
TAB_ROWS = 52000
ACC_ROWS = 50048
ACC_PT = ACC_ROWS // NS

SEG2_CHUNKS = -(-EU // (NS * CH))
SEG2_PAIRS = 3
GRP2 = 4
SEG2_PT = SEG2_CHUNKS * CH
SEG2_PAD = NS * SEG2_PT
UACC_ROWS = 10048
UACC_PT = UACC_ROWS // NS
BPT = B // NS


def _dense_body(vp, ap, tp, vf, af, tf, vw, aw, tw, vb, ab, tb,
                vcw, acw, tcw, out_lo, out_hi):
    pid = pl.program_id(0)

    def emit(h):
        out_lo[...] = h[:, :32]
        out_hi[...] = h[:, 32:]

    @pl.when(pid < 5)
    def _():
        h = None
        for p_ref, cw_ref in ((vp, vcw), (ap, acw), (tp, tcw)):
            x = p_ref[...]
            nrm = jnp.sqrt(jnp.sum(x * x, axis=1, keepdims=True))
            xn = x / jnp.maximum(nrm, 1e-12)
            hm = jnp.dot(xn, cw_ref[...], preferred_element_type=jnp.float32)
            h = hm if h is None else h + hm
        emit(h)

    @pl.when(pid >= 5)
    def _():
        h = None
        for f_ref, w_ref, b_ref, cw_ref in (
                (vf, vw, vb, vcw), (af, aw, ab, acw), (tf, tw, tb, tcw)):
            t = lax.dot_general(f_ref[...], w_ref[...],
                                (((1,), (1,)), ((), ())),
                                preferred_element_type=jnp.float32)
            t = t + b_ref[...]
            nrm = jnp.sqrt(jnp.sum(t * t, axis=1, keepdims=True))
            xn = t / jnp.maximum(nrm, 1e-12)
            hm = jnp.dot(xn, cw_ref[...], preferred_element_type=jnp.float32)
            h = hm if h is None else h + hm
        emit(h)


def _final_body(gu, pos, neg, w, pos_out, neg_out):
    ua = jnp.dot(gu[...] * (1.0 / 3.0), w[...],
                 preferred_element_type=jnp.float32)
    ut = jnp.where(ua >= 0, ua, 0.01 * ua)
    pos_out[...] = jnp.sum(ut * pos[...], axis=1) * (1.0 / 3.0)
    neg_out[...] = jnp.sum(ut * neg[...], axis=1) * (1.0 / 3.0)


def _seg1_kernel(t_lo, t_hi, ucol, icol, zrows,
                 rep_lo, rep_hi, sidx_a, didx_a, rows_a, gsem_a, ssem_a,
                 sidx_b, didx_b, rows_b, gsem_b, ssem_b, acc):
    c = lax.axis_index("c")
    s = lax.axis_index("s")

    pltpu.sync_copy(zrows, acc.at[pl.ds(s * ACC_PT, ACC_PT)])
    plsc.subcore_barrier()

    def run_edges(tab, sarr, darr):
        cbase = (s % 8) * SEG1_CHUNKS

        def stage(sidx, didx, crow):
            pltpu.sync_copy(sarr.at[pl.ds(crow, GRP)], sidx)
            pltpu.sync_copy(darr.at[pl.ds(crow, GRP)], didx)

        def fire_g(sidx, rows, gsem):
            for k in range(GRP):
                pltpu.async_copy(tab.at[sidx.at[k]],
                                 rows.at[pl.ds(k * CH, CH)], gsem)

        def drain_g(sidx, rows, gsem):
            for k in range(GRP):
                pltpu.make_async_copy(tab.at[sidx.at[k]],
                                      rows.at[pl.ds(k * CH, CH)], gsem).wait()

        def fire_s(didx, rows, ssem):
            for k in range(GRP):
                pltpu.async_copy(rows.at[pl.ds(k * CH, CH)],
                                 acc.at[didx.at[k]], ssem, add=True)

        def drain_s(didx, rows, ssem):
            for k in range(GRP):
                pltpu.make_async_copy(rows.at[pl.ds(k * CH, CH)],
                                      acc.at[didx.at[k]], ssem).wait()

        npairs = SEG1_PAIRS
        stage(sidx_a, didx_a, cbase)
        fire_g(sidx_a, rows_a, gsem_a)

        def body(p, carry):
            crow_b = cbase + (2 * p + 1) * GRP
            drain_g(sidx_a, rows_a, gsem_a)

            @pl.when(p > 0)
            def _():
                drain_s(didx_b, rows_b, ssem_b)
            fire_s(didx_a, rows_a, ssem_a)
            stage(sidx_b, didx_b, crow_b)
            fire_g(sidx_b, rows_b, gsem_b)
            drain_s(didx_a, rows_a, ssem_a)

            @pl.when(p < npairs - 1)
            def _():
                stage(sidx_a, didx_a, crow_b + GRP)
                fire_g(sidx_a, rows_a, gsem_a)
            drain_g(sidx_b, rows_b, gsem_b)
            fire_s(didx_b, rows_b, ssem_b)
            return carry
        lax.fori_loop(0, npairs, body, 0)
        drain_s(didx_b, rows_b, ssem_b)
        stage(sidx_a, didx_a, cbase + 2 * npairs * GRP)
        fire_g(sidx_a, rows_a, gsem_a)
        drain_g(sidx_a, rows_a, gsem_a)
        fire_s(didx_a, rows_a, ssem_a)
        drain_s(didx_a, rows_a, ssem_a)

    @pl.when(jnp.logical_and(c == 0, s < 8))
    def _():
        run_edges(t_lo, ucol, icol)

    @pl.when(jnp.logical_and(c == 0, s >= 8))
    def _():
        run_edges(t_lo, icol, ucol)

    @pl.when(jnp.logical_and(c == 1, s < 8))
    def _():
        run_edges(t_hi, ucol, icol)

    @pl.when(jnp.logical_and(c == 1, s >= 8))
    def _():
        run_edges(t_hi, icol, ucol)

    plsc.subcore_barrier()

    row0 = s * ACC_PT

    @pl.when(c == 0)
    def _():
        pltpu.sync_copy(acc.at[pl.ds(row0, ACC_PT)],
                        rep_lo.at[pl.ds(row0, ACC_PT)])

    @pl.when(c == 1)
    def _():
        pltpu.sync_copy(acc.at[pl.ds(row0, ACC_PT)],
                        rep_hi.at[pl.ds(row0, ACC_PT)])


def _seg2_kernel(rep_lo, rep_hi, s5ix, d5ix, posix, negix, unodes, zrows,
                 uacc_lo, uacc_hi, pos_lo, pos_hi, neg_lo, neg_hi,
                 gu_lo, gu_hi,
                 sidx_a, didx_a, rows_a, gsem_a, ssem_a,
                 sidx_b, didx_b, rows_b, gsem_b, ssem_b,
                 bidx, brows, uacc):
    c = lax.axis_index("c")
    s = lax.axis_index("s")

    pltpu.sync_copy(zrows, uacc.at[pl.ds(s * UACC_PT, UACC_PT)])
    plsc.subcore_barrier()

    def run(rep_tab, pos_out, neg_out, uacc_out, gu_out):
        cbase = s * SEG2_CHUNKS

        def stage(sidx, didx, crow):
            pltpu.sync_copy(s5ix.at[pl.ds(crow, GRP2)], sidx)
            pltpu.sync_copy(d5ix.at[pl.ds(crow, GRP2)], didx)

        def fire_g(sidx, rows, gsem):
            for k in range(GRP2):
                pltpu.async_copy(rep_tab.at[sidx.at[k]],
                                 rows.at[pl.ds(k * CH, CH)], gsem)

        def drain_g(sidx, rows, gsem):
            for k in range(GRP2):
                pltpu.make_async_copy(rep_tab.at[sidx.at[k]],
                                      rows.at[pl.ds(k * CH, CH)], gsem).wait()

        def fire_s(didx, rows, ssem):
            for k in range(GRP2):
                pltpu.async_copy(rows.at[pl.ds(k * CH, CH)],
                                 uacc.at[didx.at[k]], ssem, add=True)

        def drain_s(didx, rows, ssem):
            for k in range(GRP2):
                pltpu.make_async_copy(rows.at[pl.ds(k * CH, CH)],
                                      uacc.at[didx.at[k]], ssem).wait()

        stage(sidx_a, didx_a, cbase)
        fire_g(sidx_a, rows_a, gsem_a)

        def body(p, carry):
            crow_b = cbase + (2 * p + 1) * GRP2
            drain_g(sidx_a, rows_a, gsem_a)

            @pl.when(p > 0)
            def _():
                drain_s(didx_b, rows_b, ssem_b)
            fire_s(didx_a, rows_a, ssem_a)
            stage(sidx_b, didx_b, crow_b)
            fire_g(sidx_b, rows_b, gsem_b)
            drain_s(didx_a, rows_a, ssem_a)

            @pl.when(p < SEG2_PAIRS - 1)
            def _():
                stage(sidx_a, didx_a, crow_b + GRP2)
                fire_g(sidx_a, rows_a, gsem_a)
            drain_g(sidx_b, rows_b, gsem_b)
            fire_s(didx_b, rows_b, ssem_b)
            return carry
        lax.fori_loop(0, SEG2_PAIRS, body, 0)
        drain_s(didx_b, rows_b, ssem_b)
        pltpu.sync_copy(s5ix.at[pl.ds(cbase + 24, 1)],
                        sidx_a.at[pl.ds(0, 1)])
        pltpu.sync_copy(d5ix.at[pl.ds(cbase + 24, 1)],
                        didx_a.at[pl.ds(0, 1)])
        pltpu.sync_copy(rep_tab.at[sidx_a.at[0]], rows_a.at[pl.ds(0, CH)])
        pltpu.sync_copy(rows_a.at[pl.ds(0, CH)], uacc.at[didx_a.at[0]],
                        add=True)

        b0 = s * BPT
        pltpu.sync_copy(posix.at[pl.ds(b0, BPT)], bidx)
        pltpu.sync_copy(rep_tab.at[bidx], brows)
        pltpu.sync_copy(brows, pos_out.at[pl.ds(b0, BPT)])
        pltpu.sync_copy(negix.at[pl.ds(b0, BPT)], bidx)
        pltpu.sync_copy(rep_tab.at[bidx], brows)
        pltpu.sync_copy(brows, neg_out.at[pl.ds(b0, BPT)])

        plsc.subcore_barrier()
        r0 = s * UACC_PT
        pltpu.sync_copy(uacc.at[pl.ds(r0, UACC_PT)],
                        uacc_out.at[pl.ds(r0, UACC_PT)])
        plsc.subcore_barrier()
        pltpu.sync_copy(unodes.at[pl.ds(b0, BPT)], bidx)
        pltpu.sync_copy(uacc_out.at[bidx], brows)
        pltpu.sync_copy(brows, gu_out.at[pl.ds(b0, BPT)])

    @pl.when(c == 0)
    def _():
        run(rep_lo, pos_lo, neg_lo, uacc_lo, gu_lo)

    @pl.when(c == 1)
    def _():
        run(rep_hi, pos_hi, neg_hi, uacc_hi, gu_hi)


def _sc_mesh():
    return plsc.VectorSubcoreMesh(core_axis_name="c", subcore_axis_name="s",
                                  num_cores=NC, num_subcores=NS)


@functools.partial(jax.jit, static_argnames=())
def kernel(v_feat, a_feat, t_feat, v_pref, a_pref, t_pref, v_mlp_w, v_mlp_b,
           a_mlp_w, a_mlp_b, t_mlp_w, t_mlp_b, v_conv_w, a_conv_w, t_conv_w,
           user_conv_w, edge_index, user_index_5, user_nodes, pos_item_nodes,
           neg_item_nodes):
    f32 = jnp.float32
    i32 = jnp.int32

    bm = 2000
    vb2 = v_mlp_b.reshape(1, D_LAT)
    ab2 = a_mlp_b.reshape(1, D_LAT)
    tb2 = t_mlp_b.reshape(1, D_LAT)
    t_lo, t_hi = pl.pallas_call(
        _dense_body,
        grid=(TAB_ROWS // bm,),
        in_specs=[pl.BlockSpec((bm, D_LAT), lambda i: (jnp.minimum(i, 4), 0))] * 3
                 + [pl.BlockSpec((bm, D_FEAT),
                                 lambda i: (jnp.clip(i - 5, 0, 19), 0))] * 3
                 + [pl.BlockSpec((D_LAT, D_FEAT), lambda i: (0, 0))] * 3
                 + [pl.BlockSpec((1, D_LAT), lambda i: (0, 0))] * 3
                 + [pl.BlockSpec((D_LAT, D_LAT), lambda i: (0, 0))] * 3,
        out_specs=[pl.BlockSpec((bm, 32), lambda i: (i, 0))] * 2,
        out_shape=[jax.ShapeDtypeStruct((TAB_ROWS, 32), f32)] * 2,
    )(v_pref, a_pref, t_pref, v_feat, a_feat, t_feat,
      v_mlp_w, a_mlp_w, t_mlp_w, vb2, ab2, tb2, v_conv_w, a_conv_w, t_conv_w)

    ei = edge_index.astype(i32)
    cpad = jnp.full((SEG1_HALF - E,), TROW, i32)
    ucol = jnp.concatenate([ei[:, 0], cpad]).reshape(-1, CH)
    icol = jnp.concatenate([ei[:, 1], cpad]).reshape(-1, CH)

    zrows1 = jnp.zeros((ACC_PT, 32), f32)

    rep_lo, rep_hi = pl.kernel(
        _seg1_kernel,
        out_type=[jax.ShapeDtypeStruct((ACC_ROWS, 32), f32)] * 2,
        mesh=_sc_mesh(),
        compiler_params=pltpu.CompilerParams(use_tc_tiling_on_sc=False),
        scratch_types=[
            pltpu.VMEM((GRP, CH), i32),
            pltpu.VMEM((GRP, CH), i32),
            pltpu.VMEM((GRP * CH, 32), f32),
            pltpu.SemaphoreType.DMA,
            pltpu.SemaphoreType.DMA,
        ] * 2 + [
            pltpu.VMEM_SHARED((ACC_ROWS, 32), f32),
        ],
    )(t_lo, t_hi, ucol, icol, zrows1)

    s5 = user_index_5[0].astype(i32)
    d5 = user_index_5[1].astype(i32)
    pad2 = SEG2_PAD - EU
    s5ix = jnp.concatenate([s5, jnp.zeros((pad2,), i32)]).reshape(-1, CH)
    d5ix = jnp.concatenate([d5, jnp.full((pad2,), NUM_USER, i32)]
                           ).reshape(-1, CH)
    zrows2 = jnp.zeros((UACC_PT, 32), f32)

    (uacc_lo, uacc_hi, pos_lo, pos_hi, neg_lo, neg_hi, gu_lo, gu_hi
     ) = pl.kernel(
        _seg2_kernel,
        out_type=[jax.ShapeDtypeStruct((UACC_ROWS, 32), f32)] * 2
                 + [jax.ShapeDtypeStruct((B, 32), f32)] * 6,
        mesh=_sc_mesh(),
        compiler_params=pltpu.CompilerParams(use_tc_tiling_on_sc=False),
        scratch_types=[
            pltpu.VMEM((GRP2, CH), i32),
            pltpu.VMEM((GRP2, CH), i32),
            pltpu.VMEM((GRP2 * CH, 32), f32),
            pltpu.SemaphoreType.DMA,
            pltpu.SemaphoreType.DMA,
        ] * 2 + [
            pltpu.VMEM((BPT,), i32),
            pltpu.VMEM((BPT, 32), f32),
            pltpu.VMEM_SHARED((UACC_ROWS, 32), f32),
        ],
    )(rep_lo, rep_hi, s5ix, d5ix,
      pos_item_nodes.astype(i32), neg_item_nodes.astype(i32),
      user_nodes.astype(i32), zrows2)

    del uacc_lo, uacc_hi

    gu = jnp.concatenate([gu_lo, gu_hi], axis=1)
    pos = jnp.concatenate([pos_lo, pos_hi], axis=1)
    neg = jnp.concatenate([neg_lo, neg_hi], axis=1)

    pos_scores, neg_scores = pl.pallas_call(
        _final_body,
        out_shape=[jax.ShapeDtypeStruct((B,), f32)] * 2,
    )(gu, pos, neg, user_conv_w)

    return (pos_scores, neg_scores)


# speedup vs baseline: 1.0649x; 1.0019x over previous
import functools

import jax
import jax.numpy as jnp
from jax import lax
from jax.experimental import pallas as pl
from jax.experimental.pallas import tpu as pltpu
from jax.experimental.pallas import tpu_sc as plsc

NUM_USER = 10000
NUM_ITEM = 40000
N = NUM_USER + NUM_ITEM
E = 500000
EU = 50000
D_FEAT = 128
D_LAT = 64
B = 1024

NC = 2
NS = 16
CH = 128

GRP = 3
SEG1_CHUNKS = 489
SEG1_PT = SEG1_CHUNKS * CH
SEG1_HALF = 8 * SEG1_PT
SEG1_PAIRS = 81
TROW = 50000
TAB_ROWS = 52000
ACC_ROWS = 50048
ACC_PT = ACC_ROWS // NS

SEG2_CHUNKS = -(-EU // (NS * CH))
SEG2_PAIRS = 3
GRP2 = 4
SEG2_PT = SEG2_CHUNKS * CH
SEG2_PAD = NS * SEG2_PT
UACC_ROWS = 10048
UACC_PT = UACC_ROWS // NS
BPT = B // NS


def _dense_body(vp, ap, tp, vf, af, tf, vw, aw, tw, vb, ab, tb,
                vcw, acw, tcw, out_lo, out_hi):
    pid = pl.program_id(0)

    def emit(h):
        out_lo[...] = h[:, :32]
        out_hi[...] = h[:, 32:]

    @pl.when(pid < 5)
    def _():
        h = None
        for p_ref, cw_ref in ((vp, vcw), (ap, acw), (tp, tcw)):
            x = p_ref[...]
            nrm = jnp.sqrt(jnp.sum(x * x, axis=1, keepdims=True))
            xn = x / jnp.maximum(nrm, 1e-12)
            hm = jnp.dot(xn, cw_ref[...], preferred_element_type=jnp.float32)
            h = hm if h is None else h + hm
        emit(h)

    @pl.when(pid >= 5)
    def _():
        h = None
        for f_ref, w_ref, b_ref, cw_ref in (
                (vf, vw, vb, vcw), (af, aw, ab, acw), (tf, tw, tb, tcw)):
            t = lax.dot_general(f_ref[...], w_ref[...],
                                (((1,), (1,)), ((), ())),
                                preferred_element_type=jnp.float32)
            t = t + b_ref[...]
            nrm = jnp.sqrt(jnp.sum(t * t, axis=1, keepdims=True))
            xn = t / jnp.maximum(nrm, 1e-12)
            hm = jnp.dot(xn, cw_ref[...], preferred_element_type=jnp.float32)
            h = hm if h is None else h + hm
        emit(h)


def _final_body(gu, pos, neg, w, pos_out, neg_out):
    ua = jnp.dot(gu[...] * (1.0 / 3.0), w[...],
                 preferred_element_type=jnp.float32)
    ut = jnp.where(ua >= 0, ua, 0.01 * ua)
    pos_out[...] = jnp.sum(ut * pos[...], axis=1) * (1.0 / 3.0)
    neg_out[...] = jnp.sum(ut * neg[...], axis=1) * (1.0 / 3.0)


def _seg1_kernel(t_lo, t_hi, ucol, icol, zrows,
                 rep_lo, rep_hi, sidx_a, didx_a, rows_a, gsem_a, ssem_a,
                 sidx_b, didx_b, rows_b, gsem_b, ssem_b, acc):
    c = lax.axis_index("c")
    s = lax.axis_index("s")

    pltpu.sync_copy(zrows, acc.at[pl.ds(s * ACC_PT, ACC_PT)])
    plsc.subcore_barrier()

    def run_edges(tab, sarr, darr):
        cbase = (s % 8) * SEG1_CHUNKS

        def stage(sidx, didx, crow):
            pltpu.sync_copy(sarr.at[pl.ds(crow * CH, GRP * CH)], sidx)
            pltpu.sync_copy(darr.at[pl.ds(crow * CH, GRP * CH)], didx)

        def fire_g(sidx, rows, gsem):
            for k in range(GRP):
                pltpu.async_copy(tab.at[sidx.at[pl.ds(k * CH, CH)]],
                                 rows.at[pl.ds(k * CH, CH)], gsem)

        def drain_g(sidx, rows, gsem):
            for k in range(GRP):
                pltpu.make_async_copy(tab.at[sidx.at[pl.ds(k * CH, CH)]],
                                      rows.at[pl.ds(k * CH, CH)], gsem).wait()

        def fire_s(didx, rows, ssem):
            for k in range(GRP):
                pltpu.async_copy(rows.at[pl.ds(k * CH, CH)],
                                 acc.at[didx.at[pl.ds(k * CH, CH)]], ssem,
                                 add=True)

        def drain_s(didx, rows, ssem):
            for k in range(GRP):
                pltpu.make_async_copy(rows.at[pl.ds(k * CH, CH)],
                                      acc.at[didx.at[pl.ds(k * CH, CH)]],
                                      ssem).wait()

        npairs = SEG1_PAIRS
        stage(sidx_a, didx_a, cbase)
        fire_g(sidx_a, rows_a, gsem_a)

        def body(p, carry):
            crow_b = cbase + (2 * p + 1) * GRP
            drain_g(sidx_a, rows_a, gsem_a)

            @pl.when(p > 0)
            def _():
                drain_s(didx_b, rows_b, ssem_b)
            fire_s(didx_a, rows_a, ssem_a)
            stage(sidx_b, didx_b, crow_b)
            fire_g(sidx_b, rows_b, gsem_b)
            drain_s(didx_a, rows_a, ssem_a)

            @pl.when(p < npairs - 1)
            def _():
                stage(sidx_a, didx_a, crow_b + GRP)
                fire_g(sidx_a, rows_a, gsem_a)
            drain_g(sidx_b, rows_b, gsem_b)
            fire_s(didx_b, rows_b, ssem_b)
            return carry
        lax.fori_loop(0, npairs, body, 0)
        drain_s(didx_b, rows_b, ssem_b)
        stage(sidx_a, didx_a, cbase + 2 * npairs * GRP)
        fire_g(sidx_a, rows_a, gsem_a)
        drain_g(sidx_a, rows_a, gsem_a)
        fire_s(didx_a, rows_a, ssem_a)
        drain_s(didx_a, rows_a, ssem_a)

    @pl.when(jnp.logical_and(c == 0, s < 8))
    def _():
        run_edges(t_lo, ucol, icol)

    @pl.when(jnp.logical_and(c == 0, s >= 8))
    def _():
        run_edges(t_lo, icol, ucol)

    @pl.when(jnp.logical_and(c == 1, s < 8))
    def _():
        run_edges(t_hi, ucol, icol)

    @pl.when(jnp.logical_and(c == 1, s >= 8))
    def _():
        run_edges(t_hi, icol, ucol)

    plsc.subcore_barrier()

    row0 = s * ACC_PT

    @pl.when(c == 0)
    def _():
        pltpu.sync_copy(acc.at[pl.ds(row0, ACC_PT)],
                        rep_lo.at[pl.ds(row0, ACC_PT)])

    @pl.when(c == 1)
    def _():
        pltpu.sync_copy(acc.at[pl.ds(row0, ACC_PT)],
                        rep_hi.at[pl.ds(row0, ACC_PT)])


def _seg2_kernel(rep_lo, rep_hi, s5ix, d5ix, posix, negix, unodes, zrows,
                 uacc_lo, uacc_hi, pos_lo, pos_hi, neg_lo, neg_hi,
                 gu_lo, gu_hi,
                 sidx_a, didx_a, rows_a, gsem_a, ssem_a,
                 sidx_b, didx_b, rows_b, gsem_b, ssem_b,
                 bidx, brows, uacc):
    c = lax.axis_index("c")
    s = lax.axis_index("s")

    pltpu.sync_copy(zrows, uacc.at[pl.ds(s * UACC_PT, UACC_PT)])
    plsc.subcore_barrier()

    def run(rep_tab, pos_out, neg_out, uacc_out, gu_out):
        cbase = s * SEG2_CHUNKS

        def stage(sidx, didx, crow):
            pltpu.sync_copy(s5ix.at[pl.ds(crow, GRP2)], sidx)
            pltpu.sync_copy(d5ix.at[pl.ds(crow, GRP2)], didx)

        def fire_g(sidx, rows, gsem):
            for k in range(GRP2):
                pltpu.async_copy(rep_tab.at[sidx.at[k]],
                                 rows.at[pl.ds(k * CH, CH)], gsem)

        def drain_g(sidx, rows, gsem):
            for k in range(GRP2):
                pltpu.make_async_copy(rep_tab.at[sidx.at[k]],
                                      rows.at[pl.ds(k * CH, CH)], gsem).wait()

        def fire_s(didx, rows, ssem):
            for k in range(GRP2):
                pltpu.async_copy(rows.at[pl.ds(k * CH, CH)],
                                 uacc.at[didx.at[k]], ssem, add=True)

        def drain_s(didx, rows, ssem):
            for k in range(GRP2):
                pltpu.make_async_copy(rows.at[pl.ds(k * CH, CH)],
                                      uacc.at[didx.at[k]], ssem).wait()

        stage(sidx_a, didx_a, cbase)
        fire_g(sidx_a, rows_a, gsem_a)

        def body(p, carry):
            crow_b = cbase + (2 * p + 1) * GRP2
            drain_g(sidx_a, rows_a, gsem_a)

            @pl.when(p > 0)
            def _():
                drain_s(didx_b, rows_b, ssem_b)
            fire_s(didx_a, rows_a, ssem_a)
            stage(sidx_b, didx_b, crow_b)
            fire_g(sidx_b, rows_b, gsem_b)
            drain_s(didx_a, rows_a, ssem_a)

            @pl.when(p < SEG2_PAIRS - 1)
            def _():
                stage(sidx_a, didx_a, crow_b + GRP2)
                fire_g(sidx_a, rows_a, gsem_a)
            drain_g(sidx_b, rows_b, gsem_b)
            fire_s(didx_b, rows_b, ssem_b)
            return carry
        lax.fori_loop(0, SEG2_PAIRS, body, 0)
        drain_s(didx_b, rows_b, ssem_b)
        pltpu.sync_copy(s5ix.at[pl.ds(cbase + 24, 1)],
                        sidx_a.at[pl.ds(0, 1)])
        pltpu.sync_copy(d5ix.at[pl.ds(cbase + 24, 1)],
                        didx_a.at[pl.ds(0, 1)])
        pltpu.sync_copy(rep_tab.at[sidx_a.at[0]], rows_a.at[pl.ds(0, CH)])
        pltpu.sync_copy(rows_a.at[pl.ds(0, CH)], uacc.at[didx_a.at[0]],
                        add=True)

        b0 = s * BPT
        pltpu.sync_copy(posix.at[pl.ds(b0, BPT)], bidx)
        pltpu.sync_copy(rep_tab.at[bidx], brows)
        pltpu.sync_copy(brows, pos_out.at[pl.ds(b0, BPT)])
        pltpu.sync_copy(negix.at[pl.ds(b0, BPT)], bidx)
        pltpu.sync_copy(rep_tab.at[bidx], brows)
        pltpu.sync_copy(brows, neg_out.at[pl.ds(b0, BPT)])

        plsc.subcore_barrier()
        r0 = s * UACC_PT
        pltpu.sync_copy(uacc.at[pl.ds(r0, UACC_PT)],
                        uacc_out.at[pl.ds(r0, UACC_PT)])
        plsc.subcore_barrier()
        pltpu.sync_copy(unodes.at[pl.ds(b0, BPT)], bidx)
        pltpu.sync_copy(uacc_out.at[bidx], brows)
        pltpu.sync_copy(brows, gu_out.at[pl.ds(b0, BPT)])

    @pl.when(c == 0)
    def _():
        run(rep_lo, pos_lo, neg_lo, uacc_lo, gu_lo)

    @pl.when(c == 1)
    def _():
        run(rep_hi, pos_hi, neg_hi, uacc_hi, gu_hi)


def _sc_mesh():
    return plsc.VectorSubcoreMesh(core_axis_name="c", subcore_axis_name="s",
                                  num_cores=NC, num_subcores=NS)


@functools.partial(jax.jit, static_argnames=())
def kernel(v_feat, a_feat, t_feat, v_pref, a_pref, t_pref, v_mlp_w, v_mlp_b,
           a_mlp_w, a_mlp_b, t_mlp_w, t_mlp_b, v_conv_w, a_conv_w, t_conv_w,
           user_conv_w, edge_index, user_index_5, user_nodes, pos_item_nodes,
           neg_item_nodes):
    f32 = jnp.float32
    i32 = jnp.int32

    bm = 2000
    vb2 = v_mlp_b.reshape(1, D_LAT)
    ab2 = a_mlp_b.reshape(1, D_LAT)
    tb2 = t_mlp_b.reshape(1, D_LAT)
    t_lo, t_hi = pl.pallas_call(
        _dense_body,
        grid=(TAB_ROWS // bm,),
        in_specs=[pl.BlockSpec((bm, D_LAT), lambda i: (jnp.minimum(i, 4), 0))] * 3
                 + [pl.BlockSpec((bm, D_FEAT),
                                 lambda i: (jnp.clip(i - 5, 0, 19), 0))] * 3
                 + [pl.BlockSpec((D_LAT, D_FEAT), lambda i: (0, 0))] * 3
                 + [pl.BlockSpec((1, D_LAT), lambda i: (0, 0))] * 3
                 + [pl.BlockSpec((D_LAT, D_LAT), lambda i: (0, 0))] * 3,
        out_specs=[pl.BlockSpec((bm, 32), lambda i: (i, 0))] * 2,
        out_shape=[jax.ShapeDtypeStruct((TAB_ROWS, 32), f32)] * 2,
    )(v_pref, a_pref, t_pref, v_feat, a_feat, t_feat,
      v_mlp_w, a_mlp_w, t_mlp_w, vb2, ab2, tb2, v_conv_w, a_conv_w, t_conv_w)

    ei = edge_index.astype(i32)
    cpad = jnp.full((SEG1_HALF - E,), TROW, i32)
    ucol = jnp.concatenate([ei[:, 0], cpad])
    icol = jnp.concatenate([ei[:, 1], cpad])

    zrows1 = jnp.zeros((ACC_PT, 32), f32)

    rep_lo, rep_hi = pl.kernel(
        _seg1_kernel,
        out_type=[jax.ShapeDtypeStruct((ACC_ROWS, 32), f32)] * 2,
        mesh=_sc_mesh(),
        compiler_params=pltpu.CompilerParams(use_tc_tiling_on_sc=False),
        scratch_types=[
            pltpu.VMEM((GRP * CH,), i32),
            pltpu.VMEM((GRP * CH,), i32),
            pltpu.VMEM((GRP * CH, 32), f32),
            pltpu.SemaphoreType.DMA,
            pltpu.SemaphoreType.DMA,
        ] * 2 + [
            pltpu.VMEM_SHARED((ACC_ROWS, 32), f32),
        ],
    )(t_lo, t_hi, ucol, icol, zrows1)

    s5 = user_index_5[0].astype(i32)
    d5 = user_index_5[1].astype(i32)
    pad2 = SEG2_PAD - EU
    s5ix = jnp.concatenate([s5, jnp.zeros((pad2,), i32)]).reshape(-1, CH)
    d5ix = jnp.concatenate([d5, jnp.full((pad2,), NUM_USER, i32)]
                           ).reshape(-1, CH)
    zrows2 = jnp.zeros((UACC_PT, 32), f32)

    (uacc_lo, uacc_hi, pos_lo, pos_hi, neg_lo, neg_hi, gu_lo, gu_hi
     ) = pl.kernel(
        _seg2_kernel,
        out_type=[jax.ShapeDtypeStruct((UACC_ROWS, 32), f32)] * 2
                 + [jax.ShapeDtypeStruct((B, 32), f32)] * 6,
        mesh=_sc_mesh(),
        compiler_params=pltpu.CompilerParams(use_tc_tiling_on_sc=False),
        scratch_types=[
            pltpu.VMEM((GRP2, CH), i32),
            pltpu.VMEM((GRP2, CH), i32),
            pltpu.VMEM((GRP2 * CH, 32), f32),
            pltpu.SemaphoreType.DMA,
            pltpu.SemaphoreType.DMA,
        ] * 2 + [
            pltpu.VMEM((BPT,), i32),
            pltpu.VMEM((BPT, 32), f32),
            pltpu.VMEM_SHARED((UACC_ROWS, 32), f32),
        ],
    )(rep_lo, rep_hi, s5ix, d5ix,
      pos_item_nodes.astype(i32), neg_item_nodes.astype(i32),
      user_nodes.astype(i32), zrows2)

    del uacc_lo, uacc_hi

    gu = jnp.concatenate([gu_lo, gu_hi], axis=1)
    pos = jnp.concatenate([pos_lo, pos_hi], axis=1)
    neg = jnp.concatenate([neg_lo, neg_hi], axis=1)

    pos_scores, neg_scores = pl.pallas_call(
        _final_body,
        out_shape=[jax.ShapeDtypeStruct((B,), f32)] * 2,
    )(gu, pos, neg, user_conv_w)

    return (pos_scores, neg_scores)
